# baseline (speedup 1.0000x reference)
import functools

import jax
import jax.numpy as jnp
from jax.experimental import pallas as pl

_THR = 0.05
_ALPHA = 0.2
_BIG_NEG = -3.0e38


def _pick_block(n, candidates):
    for b in candidates:
        if n % b == 0 and b % 8 == 0:
            return b
    return n


def _prologue_body(x_ref, w_ref, ap_ref, h_ref, s1_ref, sv_ref, mx_ref):
    i = pl.program_id(0)
    h = jnp.dot(x_ref[...], w_ref[...], preferred_element_type=jnp.float32)
    h_ref[...] = h
    s12 = jnp.dot(h, ap_ref[...], preferred_element_type=jnp.float32)
    rs = jnp.sum(h, axis=1, keepdims=True)
    s1_ref[...] = s12[:, 0:1]
    s2m = jnp.where(rs != 0.0, s12[:, 1:2], _BIG_NEG)
    vldf = jnp.where(rs != 0.0, 1.0, 0.0)
    sv_ref[...] = jnp.concatenate([s2m, vldf], axis=1)

    @pl.when(i == 0)
    def _init():
        mx_ref[...] = jnp.full((1, 1), _BIG_NEG, jnp.float32)

    mx_ref[...] = jnp.maximum(mx_ref[...], jnp.max(s2m).reshape(1, 1))


def _attn_body(adj_ref, s1_ref, aux_ref, mx_ref, h_ref, out_ref, *, block_b):
    i = pl.program_id(0)
    adjb = adj_ref[...]
    s1 = s1_ref[...]
    s2m = aux_ref[0:1, :]
    vldf = aux_ref[1:2, :]
    c0 = s1 + mx_ref[...]
    emaxc = jnp.maximum(c0, _ALPHA * c0)
    m0 = adjb > 0.0
    maskf = jnp.where(m0, vldf, 0.0)
    big_l = jnp.sum(maskf, axis=1, keepdims=True)
    pre = s1 + s2m
    e = jnp.maximum(pre, _ALPHA * pre)
    x = jnp.exp(e - emaxc)
    p = jnp.where(m0, x, 0.0)
    z = jnp.sum(p, axis=1, keepdims=True)
    pmax = jnp.max(p, axis=1, keepdims=True)
    zs = jnp.where(z > 0, z, 1.0)
    top = pmax / zs
    wl = jnp.where(top > _THR, big_l / zs, 0.0)
    thr2 = _THR * zs
    w = jnp.where(p >= thr2, p, 0.0) * wl
    hp = jnp.dot(w, h_ref[...], preferred_element_type=jnp.float32)
    hb = h_ref[pl.ds(i * block_b, block_b), :]
    y = hb + hp
    out_ref[...] = jnp.where(y > 0, y, jnp.exp(y) - 1.0)


def kernel(input, adj, M, W, a):
    x = jnp.asarray(input, jnp.float32)
    n, d_in = x.shape
    d_out = W.shape[1]
    a_pair = jnp.concatenate([a[:d_out], a[d_out:]], axis=1)

    bp = _pick_block(n, (2000, 1000, 400, 200, 80, 40, 16, 8))
    h, s1c, sv, mx = pl.pallas_call(
        _prologue_body,
        grid=(n // bp,),
        in_specs=[
            pl.BlockSpec((bp, d_in), lambda i: (i, 0)),
            pl.BlockSpec((d_in, d_out), lambda i: (0, 0)),
            pl.BlockSpec((d_out, 2), lambda i: (0, 0)),
        ],
        out_specs=[
            pl.BlockSpec((bp, d_out), lambda i: (i, 0)),
            pl.BlockSpec((bp, 1), lambda i: (i, 0)),
            pl.BlockSpec((bp, 2), lambda i: (i, 0)),
            pl.BlockSpec((1, 1), lambda i: (0, 0)),
        ],
        out_shape=[
            jax.ShapeDtypeStruct((n, d_out), jnp.float32),
            jax.ShapeDtypeStruct((n, 1), jnp.float32),
            jax.ShapeDtypeStruct((n, 2), jnp.float32),
            jax.ShapeDtypeStruct((1, 1), jnp.float32),
        ],
    )(x, W, a_pair)

    aux = jnp.transpose(sv)

    b = _pick_block(n, (200, 80, 40, 16, 8))
    out = pl.pallas_call(
        functools.partial(_attn_body, block_b=b),
        grid=(n // b,),
        in_specs=[
            pl.BlockSpec((b, n), lambda i: (i, 0)),
            pl.BlockSpec((b, 1), lambda i: (i, 0)),
            pl.BlockSpec((2, n), lambda i: (0, 0)),
            pl.BlockSpec((1, 1), lambda i: (0, 0)),
            pl.BlockSpec((n, d_out), lambda i: (0, 0)),
        ],
        out_specs=pl.BlockSpec((b, d_out), lambda i: (i, 0)),
        out_shape=jax.ShapeDtypeStruct((n, d_out), jnp.float32),
    )(adj, s1c, aux, mx, h)
    return out


# speedup vs baseline: 1.0129x; 1.0129x over previous
import functools

import jax
import jax.numpy as jnp
from jax.experimental import pallas as pl

_THR = 0.05
_ALPHA = 0.2
_BIG_NEG = -3.0e38


def _pick_block(n, candidates):
    for b in candidates:
        if n % b == 0 and b % 8 == 0:
            return b
    return n


def _prologue_body(x_ref, w_ref, ap_ref, h_ref, s1_ref, sv_ref, mx_ref):
    i = pl.program_id(0)
    h = jnp.dot(x_ref[...], w_ref[...], preferred_element_type=jnp.float32)
    h_ref[...] = h
    s12 = jnp.dot(h, ap_ref[...], preferred_element_type=jnp.float32)
    rs = jnp.sum(h, axis=1, keepdims=True)
    s1_ref[...] = s12[:, 0:1]
    s2m = jnp.where(rs != 0.0, s12[:, 1:2], _BIG_NEG)
    vldf = jnp.where(rs != 0.0, 1.0, 0.0)
    sv_ref[...] = jnp.concatenate([s2m, vldf], axis=1)

    @pl.when(i == 0)
    def _init():
        mx_ref[...] = jnp.full((1, 1), _BIG_NEG, jnp.float32)

    mx_ref[...] = jnp.maximum(mx_ref[...], jnp.max(s2m).reshape(1, 1))


def _attn_body(adj_ref, s1_ref, aux_ref, mx_ref, h_ref, out_ref, *, block_b):
    i = pl.program_id(0)
    adjb = adj_ref[...]
    s1 = s1_ref[...]
    s2m = aux_ref[0:1, :]
    vldf = aux_ref[1:2, :]
    c0 = s1 + mx_ref[...]
    emaxc = jnp.maximum(c0, _ALPHA * c0)
    m0 = adjb > 0.0
    maskf = jnp.where(m0, vldf, 0.0)
    big_l = jnp.sum(maskf, axis=1, keepdims=True)
    pre = s1 + s2m
    e = jnp.maximum(pre, _ALPHA * pre)
    x = jnp.exp(e - emaxc)
    p = jnp.where(m0, x, 0.0)
    z = jnp.sum(p, axis=1, keepdims=True)
    pmax = jnp.max(p, axis=1, keepdims=True)
    zs = jnp.where(z > 0, z, 1.0)
    top = pmax / zs
    wl = jnp.where(top > _THR, big_l / zs, 0.0)
    thr2 = _THR * zs
    w = jnp.where(p >= thr2, p, 0.0) * wl
    hp = jnp.dot(w, h_ref[...], preferred_element_type=jnp.float32)
    hb = h_ref[pl.ds(i * block_b, block_b), :]
    y = hb + hp
    out_ref[...] = jnp.where(y > 0, y, jnp.exp(y) - 1.0)


def kernel(input, adj, M, W, a):
    x = jnp.asarray(input, jnp.float32)
    n, d_in = x.shape
    d_out = W.shape[1]
    a_pair = jnp.concatenate([a[:d_out], a[d_out:]], axis=1)

    bp = _pick_block(n, (2000, 1000, 400, 200, 80, 40, 16, 8))
    h, s1c, sv, mx = pl.pallas_call(
        _prologue_body,
        grid=(n // bp,),
        in_specs=[
            pl.BlockSpec((bp, d_in), lambda i: (i, 0)),
            pl.BlockSpec((d_in, d_out), lambda i: (0, 0)),
            pl.BlockSpec((d_out, 2), lambda i: (0, 0)),
        ],
        out_specs=[
            pl.BlockSpec((bp, d_out), lambda i: (i, 0)),
            pl.BlockSpec((bp, 1), lambda i: (i, 0)),
            pl.BlockSpec((bp, 2), lambda i: (i, 0)),
            pl.BlockSpec((1, 1), lambda i: (0, 0)),
        ],
        out_shape=[
            jax.ShapeDtypeStruct((n, d_out), jnp.float32),
            jax.ShapeDtypeStruct((n, 1), jnp.float32),
            jax.ShapeDtypeStruct((n, 2), jnp.float32),
            jax.ShapeDtypeStruct((1, 1), jnp.float32),
        ],
    )(x, W, a_pair)

    aux = jnp.transpose(sv)

    b = _pick_block(n, (400, 200, 80, 40, 16, 8))
    out = pl.pallas_call(
        functools.partial(_attn_body, block_b=b),
        grid=(n // b,),
        in_specs=[
            pl.BlockSpec((b, n), lambda i: (i, 0)),
            pl.BlockSpec((b, 1), lambda i: (i, 0)),
            pl.BlockSpec((2, n), lambda i: (0, 0)),
            pl.BlockSpec((1, 1), lambda i: (0, 0)),
            pl.BlockSpec((n, d_out), lambda i: (0, 0)),
        ],
        out_specs=pl.BlockSpec((b, d_out), lambda i: (i, 0)),
        out_shape=jax.ShapeDtypeStruct((n, d_out), jnp.float32),
    )(adj, s1c, aux, mx, h)
    return out


# speedup vs baseline: 1.0514x; 1.0380x over previous
import functools

import jax
import jax.numpy as jnp
from jax.experimental import pallas as pl
from jax.experimental.pallas import tpu as pltpu

_THR = 0.05
_ALPHA = 0.2
_BIG_NEG = -3.0e38


def _pick_block(n, candidates):
    for b in candidates:
        if n % b == 0 and b % 8 == 0:
            return b
    return n


def _body(x_ref, w_ref, ap_ref, adj_ref, out_ref, h_s, aux_s, s1_s, *, block_b):
    i = pl.program_id(0)

    @pl.when(i == 0)
    def _prep():
        h = jnp.dot(x_ref[...], w_ref[...], preferred_element_type=jnp.float32)
        h_s[...] = h
        s12 = jnp.dot(h, ap_ref[...], preferred_element_type=jnp.float32)
        rs = jnp.sum(h, axis=1, keepdims=True)
        s1_s[...] = s12[:, 0:1]
        s2m = jnp.where(rs != 0.0, s12[:, 1:2], _BIG_NEG)
        vldf = jnp.where(rs != 0.0, 1.0, 0.0)
        aux_s[0:2, :] = jnp.transpose(jnp.concatenate([s2m, vldf], axis=1))
        aux_s[2:3, :] = jnp.broadcast_to(jnp.max(s2m), (1, aux_s.shape[1]))

    adjb = adj_ref[...]
    s1 = s1_s[pl.ds(i * block_b, block_b), :]
    s2m = aux_s[0:1, :]
    vldf = aux_s[1:2, :]
    mxv = aux_s[2:3, 0:1]
    c0 = s1 + mxv
    emaxc = jnp.maximum(c0, _ALPHA * c0)
    m0 = adjb > 0.0
    maskf = jnp.where(m0, vldf, 0.0)
    big_l = jnp.sum(maskf, axis=1, keepdims=True)
    pre = s1 + s2m
    e = jnp.maximum(pre, _ALPHA * pre)
    x = jnp.exp(e - emaxc)
    p = jnp.where(m0, x, 0.0)
    z = jnp.sum(p, axis=1, keepdims=True)
    pmax = jnp.max(p, axis=1, keepdims=True)
    zs = jnp.where(z > 0, z, 1.0)
    top = pmax / zs
    wl = jnp.where(top > _THR, big_l / zs, 0.0)
    thr2 = _THR * zs
    w = jnp.where(p >= thr2, p, 0.0) * wl
    hp = jnp.dot(w, h_s[...], preferred_element_type=jnp.float32)
    hb = h_s[pl.ds(i * block_b, block_b), :]
    y = hb + hp
    out_ref[...] = jnp.where(y > 0, y, jnp.exp(y) - 1.0)


def kernel(input, adj, M, W, a):
    x = jnp.asarray(input, jnp.float32)
    n, d_in = x.shape
    d_out = W.shape[1]
    a_pair = jnp.concatenate([a[:d_out], a[d_out:]], axis=1)

    b = _pick_block(n, (200, 80, 40, 16, 8))
    out = pl.pallas_call(
        functools.partial(_body, block_b=b),
        grid=(n // b,),
        in_specs=[
            pl.BlockSpec((n, d_in), lambda i: (0, 0)),
            pl.BlockSpec((d_in, d_out), lambda i: (0, 0)),
            pl.BlockSpec((d_out, 2), lambda i: (0, 0)),
            pl.BlockSpec((b, n), lambda i: (i, 0)),
        ],
        out_specs=pl.BlockSpec((b, d_out), lambda i: (i, 0)),
        out_shape=jax.ShapeDtypeStruct((n, d_out), jnp.float32),
        scratch_shapes=[
            pltpu.VMEM((n, d_out), jnp.float32),
            pltpu.VMEM((3, n), jnp.float32),
            pltpu.VMEM((n, 1), jnp.float32),
        ],
    )(x, W, a_pair, adj)
    return out
